# Initial kernel scaffold; baseline (speedup 1.0000x reference)
import jax
import jax.numpy as jnp
from jax.experimental import pallas as pl


def kernel(h, X, t_ij, spharms, W_rs, W_emb, W_q, W_k, ln_g, ln_b, edge_center, edge_neighbor, num_nodes):
    raise NotImplementedError("write your pallas kernel here")


# speedup vs baseline: 5.8814x; 5.8814x over previous
import jax
import jax.numpy as jnp
import numpy as np
from jax.experimental import pallas as pl

_LATENT = 64
_M = 8
_H = 16
_SPH = 9
_NW = _LATENT + 3 * _M + 3 * _M * _M
_EXP = _LATENT + _M * _SPH + _M * _M * _SPH
_BE = 1000
_BN = 1000


def _li_of(d):
    return 0 if d == 0 else (1 if d < 4 else 2)


def _build_col_index():
    idx = np.zeros(_EXP, np.int32)
    idx[:_LATENT] = np.arange(_LATENT)
    for m in range(_M):
        for d in range(_SPH):
            idx[_LATENT + m * _SPH + d] = _li_of(d) * _M + m
    base = _LATENT + _M * _SPH
    for j in range(_M):
        for i in range(_M):
            for d in range(_SPH):
                idx[base + j * _M * _SPH + i * _SPH + d] = (
                    3 * _M + _li_of(d) * _M * _M + i * _M + j)
    return idx


_COL_IDX = _build_col_index()


def _edge_body(t_ref, hj_ref, xj_ref, sph_ref, wrsx_ref, wembx_ref, wq_ref,
               wk_ref, dh_ref, x_ref, wat_ref):
    t = t_ref[...]
    hj = hj_ref[...]
    env = (jnp.dot(t, wrsx_ref[...], preferred_element_type=jnp.float32)
           * jnp.dot(hj, wembx_ref[...], preferred_element_type=jnp.float32))
    dh_ref[...] = env[:, :_LATENT]
    sph = sph_ref[...]
    s0 = _LATENT
    s1 = _LATENT + _M * _SPH
    acc = env[:, s0:s1] * jnp.concatenate([sph] * _M, axis=1)
    xj = xj_ref[...]
    eq = None
    for j in range(_M):
        xrep = jnp.concatenate(
            [xj[:, j * _SPH:(j + 1) * _SPH]] * _M, axis=1)
        term = env[:, s1 + j * _M * _SPH:s1 + (j + 1) * _M * _SPH] * xrep
        eq = term if eq is None else eq + term
    x_ref[...] = acc + eq * (1.0 / np.sqrt(float(_M)))
    q = jnp.dot(t, wq_ref[...], preferred_element_type=jnp.float32)
    k = jnp.dot(hj, wk_ref[...], preferred_element_type=jnp.float32)
    qk = q * k
    sel = (jax.lax.broadcasted_iota(jnp.int32, (_M * _H, _M), 0) // _H
           == jax.lax.broadcasted_iota(jnp.int32, (_M * _H, _M), 1)
           ).astype(jnp.float32)
    wat_ref[...] = jnp.dot(qk, sel, preferred_element_type=jnp.float32) * 4.0


def _edge_compute(t_ij, h_j, x_j, spharms, wrsx, wembx, wq, wk):
    e = t_ij.shape[0]
    eb = lambda i: (i, 0)
    wb = lambda i: (0, 0)
    return pl.pallas_call(
        _edge_body,
        grid=(e // _BE,),
        in_specs=[
            pl.BlockSpec((_BE, _LATENT), eb),
            pl.BlockSpec((_BE, _LATENT), eb),
            pl.BlockSpec((_BE, _M * _SPH), eb),
            pl.BlockSpec((_BE, _SPH), eb),
            pl.BlockSpec((_LATENT, _EXP), wb),
            pl.BlockSpec((_LATENT, _EXP), wb),
            pl.BlockSpec((_LATENT, _M * _H), wb),
            pl.BlockSpec((_LATENT, _M * _H), wb),
        ],
        out_specs=[
            pl.BlockSpec((_BE, _LATENT), eb),
            pl.BlockSpec((_BE, _M * _SPH), eb),
            pl.BlockSpec((_BE, _M), eb),
        ],
        out_shape=[
            jax.ShapeDtypeStruct((e, _LATENT), jnp.float32),
            jax.ShapeDtypeStruct((e, _M * _SPH), jnp.float32),
            jax.ShapeDtypeStruct((e, _M), jnp.float32),
        ],
    )(t_ij, h_j, x_j, spharms, wrsx, wembx, wq, wk)


def _ln_body(h_ref, dh_ref, g_ref, b_ref, o_ref):
    v = h_ref[...] + dh_ref[...]
    mu = jnp.mean(v, axis=-1, keepdims=True)
    var = jnp.mean((v - mu) ** 2, axis=-1, keepdims=True)
    o_ref[...] = (v - mu) / jnp.sqrt(var + 1e-5) * g_ref[...] + b_ref[...]


def _ln_apply(h, dh, g, b):
    n = h.shape[0]
    nb = lambda i: (i, 0)
    wb = lambda i: (0, 0)
    return pl.pallas_call(
        _ln_body,
        grid=(n // _BN,),
        in_specs=[
            pl.BlockSpec((_BN, _LATENT), nb),
            pl.BlockSpec((_BN, _LATENT), nb),
            pl.BlockSpec((1, _LATENT), wb),
            pl.BlockSpec((1, _LATENT), wb),
        ],
        out_specs=pl.BlockSpec((_BN, _LATENT), nb),
        out_shape=jax.ShapeDtypeStruct((n, _LATENT), jnp.float32),
    )(h, dh, g.reshape(1, _LATENT), b.reshape(1, _LATENT))


def _so3_body(x_ref, dx_ref, o_ref):
    v = x_ref[...] + dx_ref[...]
    sq = v * v
    out = []
    for (a, b) in ((0, 1), (1, 4), (4, 9)):
        ss = None
        for m in range(_M):
            s = jnp.sum(sq[:, m * _SPH + a:m * _SPH + b], axis=-1,
                        keepdims=True)
            ss = s if ss is None else ss + s
        inv_n = jax.lax.rsqrt(ss * (1.0 / _M) + 1e-6)
        out.append((a, b, inv_n))
    for m in range(_M):
        for (a, b, inv_n) in out:
            o_ref[:, m * _SPH + a:m * _SPH + b] = (
                v[:, m * _SPH + a:m * _SPH + b] * inv_n)


def _so3_apply(x, dx):
    n = x.shape[0]
    nb = lambda i: (i, 0)
    return pl.pallas_call(
        _so3_body,
        grid=(n // _BN,),
        in_specs=[
            pl.BlockSpec((_BN, _M * _SPH), nb),
            pl.BlockSpec((_BN, _M * _SPH), nb),
        ],
        out_specs=pl.BlockSpec((_BN, _M * _SPH), nb),
        out_shape=jax.ShapeDtypeStruct((n, _M * _SPH), jnp.float32),
    )(x, dx)


def kernel(h, X, t_ij, spharms, W_rs, W_emb, W_q, W_k, ln_g, ln_b,
           edge_center, edge_neighbor, num_nodes):
    ns = h.shape[0]
    seg = edge_center.astype(jnp.int32)
    nbr = edge_neighbor.astype(jnp.int32)
    col_idx = jnp.asarray(_COL_IDX)
    L = W_rs.shape[0]
    Xf = X.reshape(ns, _M * _SPH)
    for l in range(L):
        wrsx = jnp.take(W_rs[l], col_idx, axis=1)
        wembx = jnp.take(W_emb[l], col_idx, axis=1)
        h_j = jnp.take(h, nbr, axis=0)
        x_j = jnp.take(Xf, nbr, axis=0)
        dh_e, x_e, wat = _edge_compute(
            t_ij, h_j, x_j, spharms, wrsx, wembx, W_q[l], W_k[l])
        dh = jax.ops.segment_sum(dh_e, seg, num_segments=ns)
        h = _ln_apply(h, dh, ln_g[l], ln_b[l])
        mx = jax.ops.segment_max(wat, seg, num_segments=ns)
        ex = jnp.exp(wat - jnp.take(mx, seg, axis=0))
        sm = jax.ops.segment_sum(ex, seg, num_segments=ns)
        alpha = ex / (jnp.take(sm, seg, axis=0) + 1e-16)
        xw = x_e * jnp.repeat(alpha, _SPH, axis=1)
        dx = jax.ops.segment_sum(xw, seg, num_segments=ns)
        Xf = _so3_apply(Xf, dx)
    return (h, Xf.reshape(ns, _M, _SPH))
